# baseline (speedup 1.0000x reference)
import functools

import jax
import jax.numpy as jnp
from jax import lax
from jax.experimental import pallas as pl
from jax.experimental.pallas import tpu as pltpu
from jax.experimental.pallas import tpu_sc as plsc

N = 10000
E = 320000
D = 128
L = 2
T = 2
M = 10240
RPS = M // 16
BLK = 128
NBLK = E // BLK
NW = 32
QB = 80
NBLKP = NW * QB
EPAD = NBLKP * BLK - E
INNER = 10

@functools.cache
def _mesh():
    return plsc.VectorSubcoreMesh(core_axis_name="c", subcore_axis_name="s")


def _conv_body(y_hbm, src_hbm, dst_hbm, out_hbm, acc, sidx_all, didx_all,
               dblk, rows_a, rows_b, sem_a, sem_b, sem_i):
    c = lax.axis_index("c")
    s = lax.axis_index("s")
    wid = c * 16 + s
    base_e = wid * QB * BLK

    nch = RPS // BLK
    for k in range(nch):
        r0 = s * RPS + k * BLK
        pltpu.sync_copy(y_hbm.at[pl.ds(r0, BLK)], rows_a)
        pltpu.sync_copy(rows_a, acc.at[pl.ds(r0, BLK)])
    plsc.subcore_barrier()

    HQ = QB // 2
    HE = HQ * BLK

    def blk(b, carry):
        for i in range(BLK // 16):
            dblk[pl.ds(i * 16, 16)] = didx_all[pl.ds(b * BLK + i * 16, 16)]
        g = pltpu.async_copy(
            y_hbm.at[sidx_all.at[pl.ds(b * BLK, BLK)]], rows_a, sem_a)
        g.wait()
        pltpu.sync_copy(rows_a, acc.at[dblk], add=True)
        return carry

    for half in range(2):
        pltpu.sync_copy(src_hbm.at[pl.ds(base_e + half * HE, HE)], sidx_all)
        pltpu.sync_copy(dst_hbm.at[pl.ds(base_e + half * HE, HE)], didx_all)
        lax.fori_loop(0, HQ, blk, 0)
    plsc.subcore_barrier()

    for k in range(nch):
        r0 = s * RPS + k * BLK
        pltpu.sync_copy(acc.at[pl.ds(r0, BLK)], rows_a)
        pltpu.sync_copy(rows_a, out_hbm.at[c, pl.ds(r0, BLK)])


@jax.jit
def _conv_sc(y, src_blk, dst_blk):
    return pl.kernel(
        _conv_body,
        out_type=jax.ShapeDtypeStruct((2, M, D), jnp.float32),
        mesh=_mesh(),
        scratch_types=[
            pltpu.VMEM_SHARED((M, D), jnp.float32),
            pltpu.VMEM((QB // 2 * BLK,), jnp.int32),
            pltpu.VMEM((QB // 2 * BLK,), jnp.int32),
            pltpu.VMEM((BLK,), jnp.int32),
            pltpu.VMEM((BLK, D), jnp.float32),
            pltpu.VMEM((BLK, D), jnp.float32),
            pltpu.SemaphoreType.DMA,
            pltpu.SemaphoreType.DMA,
            pltpu.SemaphoreType.DMA,
        ],
    )(y, src_blk, dst_blk)


def _deg_body(dst0_hbm, dst1_hbm, out_hbm, acc0, acc1, ones_v, didx, iov):
    c = lax.axis_index("c")
    s = lax.axis_index("s")
    wid = c * 16 + s

    for i in range(BLK):
        ones_v[i] = jnp.ones((16,), jnp.float32)
        iov[i] = jnp.zeros((16,), jnp.float32)
    for k in range(RPS // BLK):
        r0 = s * RPS + k * BLK
        pltpu.sync_copy(iov, acc0.at[pl.ds(r0, BLK)])
        pltpu.sync_copy(iov, acc1.at[pl.ds(r0, BLK)])
    plsc.subcore_barrier()

    q, r = divmod(NBLK, NW)
    nblk = q + jnp.where(wid < r, 1, 0)
    base = wid * q + jnp.minimum(wid, r)

    def blk0(j, carry):
        off = (base + j) * BLK
        pltpu.sync_copy(dst0_hbm.at[pl.ds(off, BLK)], didx)
        pltpu.sync_copy(ones_v, acc0.at[didx], add=True)
        return carry

    def blk1(j, carry):
        off = (base + j) * BLK
        pltpu.sync_copy(dst1_hbm.at[pl.ds(off, BLK)], didx)
        pltpu.sync_copy(ones_v, acc1.at[didx], add=True)
        return carry

    lax.fori_loop(0, nblk, blk0, 0)
    lax.fori_loop(0, nblk, blk1, 0)
    plsc.subcore_barrier()

    for k in range(RPS // BLK):
        r0 = s * RPS + k * BLK
        pltpu.sync_copy(acc0.at[pl.ds(r0, BLK)], iov)
        pltpu.sync_copy(iov, out_hbm.at[c, 0, pl.ds(r0, BLK)])
        pltpu.sync_copy(acc1.at[pl.ds(r0, BLK)], iov)
        pltpu.sync_copy(iov, out_hbm.at[c, 1, pl.ds(r0, BLK)])


@jax.jit
def _deg_sc(dst0, dst1):
    return pl.kernel(
        _deg_body,
        out_type=jax.ShapeDtypeStruct((2, 2, M, 16), jnp.float32),
        mesh=_mesh(),
        scratch_types=[
            pltpu.VMEM_SHARED((M, 16), jnp.float32),
            pltpu.VMEM_SHARED((M, 16), jnp.float32),
            pltpu.VMEM((BLK, 16), jnp.float32),
            pltpu.VMEM((BLK,), jnp.int32),
            pltpu.VMEM((BLK, 16), jnp.float32),
        ],
    )(dst0, dst1)


def _dinv_body(da_ref, out_ref):
    d0 = da_ref[0, 0] + da_ref[1, 0]
    out_ref[0] = lax.rsqrt(jnp.sum(d0, axis=1, keepdims=True) + 1.0)
    d1 = da_ref[0, 1] + da_ref[1, 1]
    out_ref[1] = lax.rsqrt(jnp.sum(d1, axis=1, keepdims=True) + 1.0)


@jax.jit
def _dinv_tc(degacc):
    return pl.pallas_call(
        _dinv_body,
        out_shape=jax.ShapeDtypeStruct((2, M, 1), jnp.float32),
    )(degacc)


MR = M // BLK


def _row_of(I128, col):
    return jnp.sum(I128 * col, axis=0, keepdims=True)


def _col_of(I128, row):
    return jnp.sum(I128 * row, axis=1, keepdims=True)


def _dense_common(h, dinvc, p_col, Wl, wih, whh, bih, bhh, wout_ref, yout_ref):
    f32 = jnp.float32
    i32 = jnp.int32
    K = jnp.float32(D)

    inv = lax.rsqrt(jnp.sum(p_col * p_col))
    scol = jnp.dot(h, p_col, preferred_element_type=f32) * inv

    bcol = lax.bitcast_convert_type(scol, i32)
    skey = bcol ^ ((bcol >> 31) & jnp.int32(0x7FFFFFFF))
    ridc = lax.broadcasted_iota(i32, (M, 1), 0)
    skey = jnp.where(ridc < N, skey, jnp.int32(-2**31))
    ucol = lax.bitcast_convert_type(skey ^ jnp.int32(-2**31), jnp.uint32)

    u2 = ucol.reshape(MR, BLK)

    t = jnp.uint32(0)
    for k in range(15, -1, -1):
        b1 = jnp.uint32(1 << (2 * k + 1))
        b2 = jnp.uint32(1 << (2 * k))
        n1 = jnp.sum((u2 >= (t | b1)).astype(f32))
        n2 = jnp.sum((u2 >= (t | b2)).astype(f32))
        n3 = jnp.sum((u2 >= (t | b1 | b2)).astype(f32))
        t = jnp.where(n1 >= K, t | b1, t)
        nn = jnp.where(n1 >= K, n3, n2)
        t = jnp.where(nn >= K, t | b2, t)

    gt = (u2 > t).astype(f32)
    eq = (u2 == t).astype(f32)
    need = K - jnp.sum(gt)

    iA = lax.broadcasted_iota(i32, (BLK, BLK), 0)
    iB = lax.broadcasted_iota(i32, (BLK, BLK), 1)
    U = (iA <= iB).astype(f32)
    I128 = (iA == iB).astype(f32)
    rA = lax.broadcasted_iota(i32, (MR, MR), 0)
    rB = lax.broadcasted_iota(i32, (MR, MR), 1)
    Ls = (rB < rA).astype(f32)
    ones_col = jnp.ones((BLK, 1), f32)

    eq_in = jnp.dot(eq, U, preferred_element_type=f32)
    eq_off = jnp.dot(Ls, jnp.dot(eq, ones_col, preferred_element_type=f32),
                     preferred_element_type=f32)
    eq_ex = eq_off + eq_in - eq
    sel = gt + eq * (eq_ex < need).astype(f32)

    s_in = jnp.dot(sel, U, preferred_element_type=f32)
    s_off = jnp.dot(Ls, jnp.dot(sel, ones_col, preferred_element_type=f32),
                    preferred_element_type=f32)
    slot = s_off + s_in - 1.0

    khiM = (ucol >> jnp.uint32(16)).astype(f32)
    kloM = (ucol & jnp.uint32(0xFFFF)).astype(f32)
    gixM = ridc.astype(f32)

    slot_f = slot.reshape(1, M)
    sel_f = sel.reshape(1, M)
    sS = lax.broadcasted_iota(i32, (BLK, M), 0).astype(f32)
    P = (sS == slot_f).astype(f32) * sel_f

    A = jnp.concatenate([h, khiM, kloM, gixM], axis=1)
    Xa = jnp.dot(P, A, preferred_element_type=f32)

    h_sel = Xa[:, :D]
    kh = Xa[:, D:D + 1]
    kl = Xa[:, D + 1:D + 2]
    gx = Xa[:, D + 2:D + 3]
    kh_r = _row_of(I128, kh)
    kl_r = _row_of(I128, kl)
    gx_r = _row_of(I128, gx)

    beats = jnp.logical_or(kh_r > kh, jnp.logical_and(kh_r == kh, kl_r > kl))
    eqk = jnp.logical_and(kh_r == kh, kl_r == kl)
    tie = jnp.logical_and(eqk, gx_r < gx)
    rank = jnp.sum(beats.astype(f32) + tie.astype(f32), axis=1, keepdims=True)

    rank_r = _row_of(I128, rank)
    sQ = lax.broadcasted_iota(i32, (BLK, BLK), 0).astype(f32)
    Q = (sQ == rank_r).astype(f32)
    Xs = jnp.dot(Q, Xa, preferred_element_type=f32)

    ui = (Xs[:, D:D + 1].astype(i32) << 16) | Xs[:, D + 1:D + 2].astype(i32)
    sk = ui ^ jnp.int32(-2**31)
    sb = jnp.where(sk < 0, sk ^ jnp.int32(0x7FFFFFFF), sk)
    vals = lax.bitcast_convert_type(sb, f32)
    X_tilde = Xs[:, :D] * jnp.tanh(vals)

    gi = jnp.dot(X_tilde, wih.T, preferred_element_type=f32) + bih
    gh = jnp.dot(Wl, whh.T, preferred_element_type=f32) + bhh
    r_ = jax.nn.sigmoid(gi[:, :D] + gh[:, :D])
    z_ = jax.nn.sigmoid(gi[:, D:2 * D] + gh[:, D:2 * D])
    n_ = jnp.tanh(gi[:, 2 * D:] + r_ * gh[:, 2 * D:])
    w_new = (1.0 - z_) * n_ + z_ * Wl

    wout_ref[...] = w_new
    yout_ref[...] = jnp.dot(h, w_new, preferred_element_type=f32) * dinvc


def _dense_a_body(h_ref, dinvc_ref, W_ref, p_ref, wih_ref, whh_ref,
                  bih_ref, bhh_ref, wout_ref, yout_ref):
    _dense_common(h_ref[...], dinvc_ref[...], p_ref[...], W_ref[...],
                  wih_ref[...], whh_ref[...], bih_ref[...], bhh_ref[...],
                  wout_ref, yout_ref)


def _dense_b_body(acc_ref, yprev_ref, dinvp_ref, dinvc_ref, W_ref, p_ref,
                  wih_ref, whh_ref, bih_ref, bhh_ref, wout_ref, yout_ref):
    h = dinvp_ref[...] * (acc_ref[0] + acc_ref[1] - yprev_ref[...])
    _dense_common(h, dinvc_ref[...], p_ref[...], W_ref[...],
                  wih_ref[...], whh_ref[...], bih_ref[...], bhh_ref[...],
                  wout_ref, yout_ref)


_dense_out = (jax.ShapeDtypeStruct((D, D), jnp.float32),
              jax.ShapeDtypeStruct((M, D), jnp.float32))


def _dense_a(h, dinvc, Wl, p_l, wih, whh, bih, bhh):
    return pl.pallas_call(_dense_a_body, out_shape=_dense_out)(
        h, dinvc, Wl, p_l.reshape(D, 1), wih, whh,
        bih.reshape(1, 3 * D), bhh.reshape(1, 3 * D))


def _dense_b(acc, yprev, dinvp, dinvc, Wl, p_l, wih, whh, bih, bhh):
    return pl.pallas_call(_dense_b_body, out_shape=_dense_out)(
        acc, yprev, dinvp, dinvc, Wl, p_l.reshape(D, 1), wih, whh,
        bih.reshape(1, 3 * D), bhh.reshape(1, 3 * D))


def _post_body(acc_ref, y_ref, dinv_ref, out_ref):
    out_ref[...] = dinv_ref[...] * (acc_ref[0] + acc_ref[1] - y_ref[...])


def _post_tc(acc, y, dinv):
    return pl.pallas_call(
        _post_body,
        out_shape=jax.ShapeDtypeStruct((M, D), jnp.float32),
    )(acc, y, dinv)


def kernel(x_t0, x_t1, edge_index_t0, edge_index_t1, W_init, p, w_ih, w_hh, b_ih, b_hh):
    pad = ((0, M - N), (0, 0))
    xs = [jnp.pad(x_t0, pad), jnp.pad(x_t1, pad)]
    padv = (N + jnp.arange(EPAD, dtype=jnp.int32) % (M - N))

    def _blockify(v):
        return jnp.concatenate([v, padv])

    srcs = [_blockify(edge_index_t0[0]), _blockify(edge_index_t1[0])]
    dsts = [_blockify(edge_index_t0[1]), _blockify(edge_index_t1[1])]

    degacc = _deg_sc(edge_index_t0[1], edge_index_t1[1])
    dinv = _dinv_tc(degacc)

    Ws = [W_init[l] for l in range(L)]
    for t in range(T):
        Ws[0], Y = _dense_a(xs[t], dinv[t], Ws[0], p[0],
                            w_ih[0], w_hh[0], b_ih[0], b_hh[0])
        acc = _conv_sc(Y, srcs[t], dsts[t])
        Ws[1], Y = _dense_b(acc, Y, dinv[t], dinv[t], Ws[1], p[1],
                            w_ih[1], w_hh[1], b_ih[1], b_hh[1])
        if t == T - 1:
            acc = _conv_sc(Y, srcs[t], dsts[t])
    out = _post_tc(acc, Y, dinv[T - 1])
    return out[:N]


# speedup vs baseline: 1.0774x; 1.0774x over previous
import functools

import jax
import jax.numpy as jnp
from jax import lax
from jax.experimental import pallas as pl
from jax.experimental.pallas import tpu as pltpu
from jax.experimental.pallas import tpu_sc as plsc

N = 10000
E = 320000
D = 128
L = 2
T = 2
M = 10240
RPS = M // 16
BLK = 128
NBLK = E // BLK
NW = 32
QB = 80
NBLKP = NW * QB
EPAD = NBLKP * BLK - E
INNER = 10

@functools.cache
def _mesh():
    return plsc.VectorSubcoreMesh(core_axis_name="c", subcore_axis_name="s")


def _conv_body(y_hbm, src_hbm, dst_hbm, out_hbm, acc, sidx_all, didx_all,
               dblk, rows_a, rows_b, sem_a, sem_b, sem_i):
    c = lax.axis_index("c")
    s = lax.axis_index("s")
    wid = c * 16 + s
    base_e = wid * QB * BLK

    nch = RPS // BLK
    for k in range(nch):
        r0 = s * RPS + k * BLK
        pltpu.sync_copy(y_hbm.at[pl.ds(r0, BLK)], rows_a)
        pltpu.sync_copy(rows_a, acc.at[pl.ds(r0, BLK)])
    plsc.subcore_barrier()

    HQ = QB // 2
    HE = HQ * BLK

    def blk(b, carry):
        for i in range(BLK // 16):
            dblk[pl.ds(i * 16, 16)] = didx_all[pl.ds(b * BLK + i * 16, 16)]
        g = pltpu.async_copy(
            y_hbm.at[sidx_all.at[pl.ds(b * BLK, BLK)]], rows_a, sem_a)
        g.wait()
        pltpu.sync_copy(rows_a, acc.at[dblk], add=True)
        return carry

    for half in range(2):
        pltpu.sync_copy(src_hbm.at[pl.ds(base_e + half * HE, HE)], sidx_all)
        pltpu.sync_copy(dst_hbm.at[pl.ds(base_e + half * HE, HE)], didx_all)
        lax.fori_loop(0, HQ, blk, 0)
    plsc.subcore_barrier()

    for k in range(nch):
        r0 = s * RPS + k * BLK
        pltpu.sync_copy(acc.at[pl.ds(r0, BLK)], rows_a)
        pltpu.sync_copy(rows_a, out_hbm.at[c, pl.ds(r0, BLK)])


@jax.jit
def _conv_sc(y, src_blk, dst_blk):
    return pl.kernel(
        _conv_body,
        out_type=jax.ShapeDtypeStruct((2, M, D), jnp.float32),
        mesh=_mesh(),
        scratch_types=[
            pltpu.VMEM_SHARED((M, D), jnp.float32),
            pltpu.VMEM((QB // 2 * BLK,), jnp.int32),
            pltpu.VMEM((QB // 2 * BLK,), jnp.int32),
            pltpu.VMEM((BLK,), jnp.int32),
            pltpu.VMEM((BLK, D), jnp.float32),
            pltpu.VMEM((BLK, D), jnp.float32),
            pltpu.SemaphoreType.DMA,
            pltpu.SemaphoreType.DMA,
            pltpu.SemaphoreType.DMA,
        ],
    )(y, src_blk, dst_blk)


def _deg_body(dst0_hbm, dst1_hbm, out_hbm, acc0, acc1, ones_v, didx, iov,
              d0_all, d1_all):
    c = lax.axis_index("c")
    s = lax.axis_index("s")
    wid = c * 16 + s
    base_e = wid * QB * BLK

    for i in range(BLK):
        ones_v[i] = jnp.ones((16,), jnp.float32)
        iov[i] = jnp.zeros((16,), jnp.float32)
    for k in range(RPS // BLK):
        r0 = s * RPS + k * BLK
        pltpu.sync_copy(iov, acc0.at[pl.ds(r0, BLK)])
        pltpu.sync_copy(iov, acc1.at[pl.ds(r0, BLK)])
    pltpu.sync_copy(dst0_hbm.at[pl.ds(base_e, QB * BLK)], d0_all)
    pltpu.sync_copy(dst1_hbm.at[pl.ds(base_e, QB * BLK)], d1_all)
    plsc.subcore_barrier()

    def blk(b, carry):
        for i in range(BLK // 16):
            didx[pl.ds(i * 16, 16)] = d0_all[pl.ds(b * BLK + i * 16, 16)]
        pltpu.sync_copy(ones_v, acc0.at[didx], add=True)
        for i in range(BLK // 16):
            didx[pl.ds(i * 16, 16)] = d1_all[pl.ds(b * BLK + i * 16, 16)]
        pltpu.sync_copy(ones_v, acc1.at[didx], add=True)
        return carry

    lax.fori_loop(0, QB, blk, 0)
    plsc.subcore_barrier()

    for k in range(RPS // BLK):
        r0 = s * RPS + k * BLK
        pltpu.sync_copy(acc0.at[pl.ds(r0, BLK)], iov)
        pltpu.sync_copy(iov, out_hbm.at[c, 0, pl.ds(r0, BLK)])
        pltpu.sync_copy(acc1.at[pl.ds(r0, BLK)], iov)
        pltpu.sync_copy(iov, out_hbm.at[c, 1, pl.ds(r0, BLK)])


@jax.jit
def _deg_sc(dst0, dst1):
    return pl.kernel(
        _deg_body,
        out_type=jax.ShapeDtypeStruct((2, 2, M, 16), jnp.float32),
        mesh=_mesh(),
        scratch_types=[
            pltpu.VMEM_SHARED((M, 16), jnp.float32),
            pltpu.VMEM_SHARED((M, 16), jnp.float32),
            pltpu.VMEM((BLK, 16), jnp.float32),
            pltpu.VMEM((BLK,), jnp.int32),
            pltpu.VMEM((BLK, 16), jnp.float32),
            pltpu.VMEM((QB * BLK,), jnp.int32),
            pltpu.VMEM((QB * BLK,), jnp.int32),
        ],
    )(dst0, dst1)


def _dinv_body(da_ref, out_ref):
    d0 = da_ref[0, 0] + da_ref[1, 0]
    out_ref[0] = lax.rsqrt(jnp.sum(d0, axis=1, keepdims=True) + 1.0)
    d1 = da_ref[0, 1] + da_ref[1, 1]
    out_ref[1] = lax.rsqrt(jnp.sum(d1, axis=1, keepdims=True) + 1.0)


@jax.jit
def _dinv_tc(degacc):
    return pl.pallas_call(
        _dinv_body,
        out_shape=jax.ShapeDtypeStruct((2, M, 1), jnp.float32),
    )(degacc)


MR = M // BLK


def _row_of(I128, col):
    return jnp.sum(I128 * col, axis=0, keepdims=True)


def _col_of(I128, row):
    return jnp.sum(I128 * row, axis=1, keepdims=True)


def _dense_common(h, dinvc, p_col, Wl, wih, whh, bih, bhh, wout_ref, yout_ref):
    f32 = jnp.float32
    i32 = jnp.int32
    K = jnp.float32(D)

    inv = lax.rsqrt(jnp.sum(p_col * p_col))
    scol = jnp.dot(h, p_col, preferred_element_type=f32) * inv

    bcol = lax.bitcast_convert_type(scol, i32)
    skey = bcol ^ ((bcol >> 31) & jnp.int32(0x7FFFFFFF))
    ridc = lax.broadcasted_iota(i32, (M, 1), 0)
    skey = jnp.where(ridc < N, skey, jnp.int32(-2**31))
    ucol = lax.bitcast_convert_type(skey ^ jnp.int32(-2**31), jnp.uint32)

    u2 = ucol.reshape(MR, BLK)

    t = jnp.uint32(0)
    for k in range(15, -1, -1):
        b1 = jnp.uint32(1 << (2 * k + 1))
        b2 = jnp.uint32(1 << (2 * k))
        n1 = jnp.sum((u2 >= (t | b1)).astype(f32))
        n2 = jnp.sum((u2 >= (t | b2)).astype(f32))
        n3 = jnp.sum((u2 >= (t | b1 | b2)).astype(f32))
        t = jnp.where(n1 >= K, t | b1, t)
        nn = jnp.where(n1 >= K, n3, n2)
        t = jnp.where(nn >= K, t | b2, t)

    gt = (u2 > t).astype(f32)
    eq = (u2 == t).astype(f32)
    need = K - jnp.sum(gt)

    iA = lax.broadcasted_iota(i32, (BLK, BLK), 0)
    iB = lax.broadcasted_iota(i32, (BLK, BLK), 1)
    U = (iA <= iB).astype(f32)
    I128 = (iA == iB).astype(f32)
    rA = lax.broadcasted_iota(i32, (MR, MR), 0)
    rB = lax.broadcasted_iota(i32, (MR, MR), 1)
    Ls = (rB < rA).astype(f32)
    ones_col = jnp.ones((BLK, 1), f32)

    eq_in = jnp.dot(eq, U, preferred_element_type=f32)
    eq_off = jnp.dot(Ls, jnp.dot(eq, ones_col, preferred_element_type=f32),
                     preferred_element_type=f32)
    eq_ex = eq_off + eq_in - eq
    sel = gt + eq * (eq_ex < need).astype(f32)

    s_in = jnp.dot(sel, U, preferred_element_type=f32)
    s_off = jnp.dot(Ls, jnp.dot(sel, ones_col, preferred_element_type=f32),
                    preferred_element_type=f32)
    slot = s_off + s_in - 1.0

    khiM = (ucol >> jnp.uint32(16)).astype(f32)
    kloM = (ucol & jnp.uint32(0xFFFF)).astype(f32)
    gixM = ridc.astype(f32)

    slot_f = slot.reshape(1, M)
    sel_f = sel.reshape(1, M)
    sS = lax.broadcasted_iota(i32, (BLK, M), 0).astype(f32)
    P = (sS == slot_f).astype(f32) * sel_f

    A = jnp.concatenate([h, khiM, kloM, gixM], axis=1)
    Xa = jnp.dot(P, A, preferred_element_type=f32)

    h_sel = Xa[:, :D]
    kh = Xa[:, D:D + 1]
    kl = Xa[:, D + 1:D + 2]
    gx = Xa[:, D + 2:D + 3]
    kh_r = _row_of(I128, kh)
    kl_r = _row_of(I128, kl)
    gx_r = _row_of(I128, gx)

    beats = jnp.logical_or(kh_r > kh, jnp.logical_and(kh_r == kh, kl_r > kl))
    eqk = jnp.logical_and(kh_r == kh, kl_r == kl)
    tie = jnp.logical_and(eqk, gx_r < gx)
    rank = jnp.sum(beats.astype(f32) + tie.astype(f32), axis=1, keepdims=True)

    rank_r = _row_of(I128, rank)
    sQ = lax.broadcasted_iota(i32, (BLK, BLK), 0).astype(f32)
    Q = (sQ == rank_r).astype(f32)
    Xs = jnp.dot(Q, Xa, preferred_element_type=f32)

    ui = (Xs[:, D:D + 1].astype(i32) << 16) | Xs[:, D + 1:D + 2].astype(i32)
    sk = ui ^ jnp.int32(-2**31)
    sb = jnp.where(sk < 0, sk ^ jnp.int32(0x7FFFFFFF), sk)
    vals = lax.bitcast_convert_type(sb, f32)
    X_tilde = Xs[:, :D] * jnp.tanh(vals)

    gi = jnp.dot(X_tilde, wih.T, preferred_element_type=f32) + bih
    gh = jnp.dot(Wl, whh.T, preferred_element_type=f32) + bhh
    r_ = jax.nn.sigmoid(gi[:, :D] + gh[:, :D])
    z_ = jax.nn.sigmoid(gi[:, D:2 * D] + gh[:, D:2 * D])
    n_ = jnp.tanh(gi[:, 2 * D:] + r_ * gh[:, 2 * D:])
    w_new = (1.0 - z_) * n_ + z_ * Wl

    wout_ref[...] = w_new
    yout_ref[...] = jnp.dot(h, w_new, preferred_element_type=f32) * dinvc


def _dense_a_body(h_ref, dinvc_ref, W_ref, p_ref, wih_ref, whh_ref,
                  bih_ref, bhh_ref, wout_ref, yout_ref):
    _dense_common(h_ref[...], dinvc_ref[...], p_ref[...], W_ref[...],
                  wih_ref[...], whh_ref[...], bih_ref[...], bhh_ref[...],
                  wout_ref, yout_ref)


def _dense_b_body(acc_ref, yprev_ref, dinvp_ref, dinvc_ref, W_ref, p_ref,
                  wih_ref, whh_ref, bih_ref, bhh_ref, wout_ref, yout_ref):
    h = dinvp_ref[...] * (acc_ref[0] + acc_ref[1] - yprev_ref[...])
    _dense_common(h, dinvc_ref[...], p_ref[...], W_ref[...],
                  wih_ref[...], whh_ref[...], bih_ref[...], bhh_ref[...],
                  wout_ref, yout_ref)


_dense_out = (jax.ShapeDtypeStruct((D, D), jnp.float32),
              jax.ShapeDtypeStruct((M, D), jnp.float32))


def _dense_a(h, dinvc, Wl, p_l, wih, whh, bih, bhh):
    return pl.pallas_call(_dense_a_body, out_shape=_dense_out)(
        h, dinvc, Wl, p_l.reshape(D, 1), wih, whh,
        bih.reshape(1, 3 * D), bhh.reshape(1, 3 * D))


def _dense_b(acc, yprev, dinvp, dinvc, Wl, p_l, wih, whh, bih, bhh):
    return pl.pallas_call(_dense_b_body, out_shape=_dense_out)(
        acc, yprev, dinvp, dinvc, Wl, p_l.reshape(D, 1), wih, whh,
        bih.reshape(1, 3 * D), bhh.reshape(1, 3 * D))


def _post_body(acc_ref, y_ref, dinv_ref, out_ref):
    out_ref[...] = dinv_ref[...] * (acc_ref[0] + acc_ref[1] - y_ref[...])


def _post_tc(acc, y, dinv):
    return pl.pallas_call(
        _post_body,
        out_shape=jax.ShapeDtypeStruct((M, D), jnp.float32),
    )(acc, y, dinv)


def kernel(x_t0, x_t1, edge_index_t0, edge_index_t1, W_init, p, w_ih, w_hh, b_ih, b_hh):
    pad = ((0, M - N), (0, 0))
    xs = [jnp.pad(x_t0, pad), jnp.pad(x_t1, pad)]
    padv = (N + jnp.arange(EPAD, dtype=jnp.int32) % (M - N))

    def _blockify(v):
        return jnp.concatenate([v, padv])

    srcs = [_blockify(edge_index_t0[0]), _blockify(edge_index_t1[0])]
    dsts = [_blockify(edge_index_t0[1]), _blockify(edge_index_t1[1])]

    degacc = _deg_sc(dsts[0], dsts[1])
    dinv = _dinv_tc(degacc)

    Ws = [W_init[l] for l in range(L)]
    for t in range(T):
        Ws[0], Y = _dense_a(xs[t], dinv[t], Ws[0], p[0],
                            w_ih[0], w_hh[0], b_ih[0], b_hh[0])
        acc = _conv_sc(Y, srcs[t], dsts[t])
        Ws[1], Y = _dense_b(acc, Y, dinv[t], dinv[t], Ws[1], p[1],
                            w_ih[1], w_hh[1], b_ih[1], b_hh[1])
        if t == T - 1:
            acc = _conv_sc(Y, srcs[t], dsts[t])
    out = _post_tc(acc, Y, dinv[T - 1])
    return out[:N]
